# Initial kernel scaffold; baseline (speedup 1.0000x reference)
import jax
import jax.numpy as jnp
from jax.experimental import pallas as pl


def kernel(x, boundaries, W_in, b_in, W_out, b_out, size_emb, pos_enc, W1, b1, W2, b2, ln_g, ln_b):
    raise NotImplementedError("write your pallas kernel here")


# speedup vs baseline: 1.9295x; 1.9295x over previous
import jax
import jax.numpy as jnp
import numpy as np
from jax.experimental import pallas as pl
from jax.experimental.pallas import tpu as pltpu

B = 4
S = 512
D = 1536
H = 12
HD = D // H
C = 256
E = 1024
THRESH = 0.9
_SCALE = 1.0 / np.sqrt(HD)


def _erf(x):
    a1, a2, a3, a4, a5 = (0.254829592, -0.284496736, 1.421413741,
                          -1.453152027, 1.061405429)
    p = 0.3275911
    sgn = jnp.where(x < 0.0, -1.0, 1.0)
    ax = jnp.abs(x)
    t = 1.0 / (1.0 + p * ax)
    poly = ((((a5 * t + a4) * t + a3) * t + a2) * t + a1) * t
    y = 1.0 - poly * jnp.exp(-ax * ax)
    return sgn * y


def _gelu_exact(x):
    return 0.5 * x * (1.0 + _erf(x * np.float32(1.0 / np.sqrt(2.0))))


def _fused_kernel(x_ref, bnd_ref, winT_ref, bin_ref, woutT_ref, bout_ref,
                  semb_ref, pos_ref, w1T_ref, b1_ref, w2T_ref, b2_ref,
                  g_ref, beta_ref, out_ref):
    xb = x_ref[0]
    m = (bnd_ref[0] > THRESH).astype(jnp.float32)

    row = jax.lax.broadcasted_iota(jnp.int32, (S, S), 0)
    col = jax.lax.broadcasted_iota(jnp.int32, (S, S), 1)
    tri_low = (col <= row).astype(jnp.float32)
    cid_col = jnp.sum(tri_low * m, axis=1, keepdims=True)
    tri_up = (row <= col).astype(jnp.float32)
    cid_lane = jnp.dot(m, tri_up, preferred_element_type=jnp.float32)

    allowed = cid_col == cid_lane

    qkv = jnp.dot(xb, winT_ref[...], preferred_element_type=jnp.float32)
    qkv = qkv + bin_ref[0]

    o_parts = []
    for h in range(H):
        qh = qkv[:, h * HD:(h + 1) * HD].astype(jnp.bfloat16)
        kh = qkv[:, D + h * HD:D + (h + 1) * HD].astype(jnp.bfloat16)
        vh = qkv[:, 2 * D + h * HD:2 * D + (h + 1) * HD].astype(jnp.bfloat16)
        s = jax.lax.dot_general(qh, kh, (((1,), (1,)), ((), ())),
                                preferred_element_type=jnp.float32) * _SCALE
        s = jnp.where(allowed, s, -1e30)
        s = s - jnp.max(s, axis=1, keepdims=True)
        e = jnp.exp(s)
        a = e / jnp.sum(e, axis=1, keepdims=True)
        oh = jnp.dot(a.astype(jnp.bfloat16), vh,
                     preferred_element_type=jnp.float32)
        o_parts.append(oh.astype(jnp.bfloat16))
    o = jnp.concatenate(o_parts, axis=1)

    out = jnp.dot(o, woutT_ref[...], preferred_element_type=jnp.float32)
    out = out + bout_ref[0]

    slot = 1.0 + jax.lax.broadcasted_iota(jnp.int32, (C, 1), 0).astype(jnp.float32)
    onehot = (slot == cid_lane).astype(jnp.float32)
    lens = jnp.sum(onehot, axis=1, keepdims=True)
    sums = jnp.dot(onehot.astype(jnp.bfloat16), out.astype(jnp.bfloat16),
                   preferred_element_type=jnp.float32)
    means = sums / jnp.maximum(lens, 1.0)

    idx = jnp.minimum(lens, float(E - 1))
    eiota = jax.lax.broadcasted_iota(jnp.int32, (1, E), 1).astype(jnp.float32)
    oneh2 = (idx == eiota).astype(jnp.bfloat16)
    sv = jnp.dot(oneh2, semb_ref[...], preferred_element_type=jnp.float32)

    r = means + sv
    r = jnp.where(lens > 0.0, r, 0.0)
    ct = r + pos_ref[...]

    h1 = jnp.dot(ct.astype(jnp.bfloat16), w1T_ref[...],
                 preferred_element_type=jnp.float32) + b1_ref[0]
    h1 = _gelu_exact(h1)
    h2 = jnp.dot(h1.astype(jnp.bfloat16), w2T_ref[...],
                 preferred_element_type=jnp.float32) + b2_ref[0]
    mu = jnp.mean(h2, axis=1, keepdims=True)
    var = jnp.mean((h2 - mu) * (h2 - mu), axis=1, keepdims=True)
    hn = (h2 - mu) * jax.lax.rsqrt(var + 1e-5) * g_ref[0] + beta_ref[0]
    out_ref[0] = hn


def kernel(x, boundaries, W_in, b_in, W_out, b_out, size_emb, pos_enc,
           W1, b1, W2, b2, ln_g, ln_b):
    xb = x.astype(jnp.bfloat16)
    winT = W_in.T.astype(jnp.bfloat16)
    woutT = W_out.T.astype(jnp.bfloat16)
    w1T = W1.T.astype(jnp.bfloat16)
    w2T = W2.T.astype(jnp.bfloat16)
    semb = size_emb.astype(jnp.bfloat16)
    bnd = boundaries.reshape(B, 1, S)
    pos = pos_enc.reshape(C, D)

    const = lambda *_: (0, 0)
    out = pl.pallas_call(
        _fused_kernel,
        grid=(B,),
        in_specs=[
            pl.BlockSpec((1, S, D), lambda b: (b, 0, 0)),
            pl.BlockSpec((1, 1, S), lambda b: (b, 0, 0)),
            pl.BlockSpec((D, 3 * D), const),
            pl.BlockSpec((1, 3 * D), const),
            pl.BlockSpec((D, D), const),
            pl.BlockSpec((1, D), const),
            pl.BlockSpec((E, D), const),
            pl.BlockSpec((C, D), const),
            pl.BlockSpec((D, 2 * D), const),
            pl.BlockSpec((1, 2 * D), const),
            pl.BlockSpec((2 * D, D), const),
            pl.BlockSpec((1, D), const),
            pl.BlockSpec((1, D), const),
            pl.BlockSpec((1, D), const),
        ],
        out_specs=pl.BlockSpec((1, C, D), lambda b: (b, 0, 0)),
        out_shape=jax.ShapeDtypeStruct((B, C, D), jnp.float32),
    )(xb, bnd, winT, b_in.reshape(1, -1), woutT, b_out.reshape(1, -1),
      semb, pos, w1T, b1.reshape(1, -1), w2T, b2.reshape(1, -1),
      ln_g.reshape(1, -1), ln_b.reshape(1, -1))
    return out


# speedup vs baseline: 2.0122x; 1.0429x over previous
import jax
import jax.numpy as jnp
import numpy as np
from jax.experimental import pallas as pl
from jax.experimental.pallas import tpu as pltpu

B = 4
S = 512
D = 1536
H = 12
HD = D // H
C = 256
E = 1024
THRESH = 0.9
_SCALE = 1.0 / np.sqrt(HD)


def _erf(x):
    a1, a2, a3, a4, a5 = (0.254829592, -0.284496736, 1.421413741,
                          -1.453152027, 1.061405429)
    p = 0.3275911
    sgn = jnp.where(x < 0.0, -1.0, 1.0)
    ax = jnp.abs(x)
    t = 1.0 / (1.0 + p * ax)
    poly = ((((a5 * t + a4) * t + a3) * t + a2) * t + a1) * t
    y = 1.0 - poly * jnp.exp(-ax * ax)
    return sgn * y


def _gelu_exact(x):
    return 0.5 * x * (1.0 + _erf(x * np.float32(1.0 / np.sqrt(2.0))))


def _fused_kernel(x_ref, bnd_ref, winT_ref, bin_ref, woutT_ref, bout_ref,
                  semb_ref, pos_ref, w1T_ref, b1_ref, w2T_ref, b2_ref,
                  g_ref, beta_ref, out_ref):
    xb = x_ref[0]
    m = (bnd_ref[0] > THRESH).astype(jnp.float32)

    row = jax.lax.broadcasted_iota(jnp.int32, (S, S), 0)
    col = jax.lax.broadcasted_iota(jnp.int32, (S, S), 1)
    tri_low = (col <= row).astype(jnp.float32)
    cid_col = jnp.sum(tri_low * m, axis=1, keepdims=True)
    tri_up = (row <= col).astype(jnp.float32)
    cid_lane = jnp.dot(m, tri_up, preferred_element_type=jnp.float32)

    allowed = cid_col == cid_lane

    allowedf = allowed.astype(jnp.float32)

    qkv = jnp.dot(xb, winT_ref[...], preferred_element_type=jnp.float32)
    qkv = qkv + bin_ref[0]
    qs = qkv[:, :D] * _SCALE

    o_parts = []
    for h in range(H):
        qh = qs[:, h * HD:(h + 1) * HD].astype(jnp.bfloat16)
        kh = qkv[:, D + h * HD:D + (h + 1) * HD].astype(jnp.bfloat16)
        vh = qkv[:, 2 * D + h * HD:2 * D + (h + 1) * HD].astype(jnp.bfloat16)
        s = jax.lax.dot_general(qh, kh, (((1,), (1,)), ((), ())),
                                preferred_element_type=jnp.float32)
        e = jnp.exp(s - jnp.max(s, axis=1, keepdims=True)) * allowedf
        a = e * (1.0 / jnp.sum(e, axis=1, keepdims=True))
        oh = jnp.dot(a.astype(jnp.bfloat16), vh,
                     preferred_element_type=jnp.float32)
        o_parts.append(oh.astype(jnp.bfloat16))
    o = jnp.concatenate(o_parts, axis=1)

    slot = 1.0 + jax.lax.broadcasted_iota(jnp.int32, (C, 1), 0).astype(jnp.float32)
    onehot = (slot == cid_lane).astype(jnp.float32)
    lens = jnp.sum(onehot, axis=1, keepdims=True)
    sums = jnp.dot(onehot.astype(jnp.bfloat16), o,
                   preferred_element_type=jnp.float32)
    means_o = sums * (1.0 / jnp.maximum(lens, 1.0))
    means = jnp.dot(means_o.astype(jnp.bfloat16), woutT_ref[...],
                    preferred_element_type=jnp.float32) + bout_ref[0]

    idx = jnp.minimum(lens, float(E - 1))
    eiota = jax.lax.broadcasted_iota(jnp.int32, (1, E), 1).astype(jnp.float32)
    oneh2 = (idx == eiota).astype(jnp.bfloat16)
    sv = jnp.dot(oneh2, semb_ref[...], preferred_element_type=jnp.float32)

    r = means + sv
    r = jnp.where(lens > 0.0, r, 0.0)
    ct = r + pos_ref[...]

    h1 = jnp.dot(ct.astype(jnp.bfloat16), w1T_ref[...],
                 preferred_element_type=jnp.float32) + b1_ref[0]
    h1 = _gelu_exact(h1)
    h2 = jnp.dot(h1.astype(jnp.bfloat16), w2T_ref[...],
                 preferred_element_type=jnp.float32) + b2_ref[0]
    mu = jnp.mean(h2, axis=1, keepdims=True)
    var = jnp.mean((h2 - mu) * (h2 - mu), axis=1, keepdims=True)
    hn = (h2 - mu) * jax.lax.rsqrt(var + 1e-5) * g_ref[0] + beta_ref[0]
    out_ref[0] = hn


def kernel(x, boundaries, W_in, b_in, W_out, b_out, size_emb, pos_enc,
           W1, b1, W2, b2, ln_g, ln_b):
    xb = x.astype(jnp.bfloat16)
    winT = W_in.T.astype(jnp.bfloat16)
    woutT = W_out.T.astype(jnp.bfloat16)
    w1T = W1.T.astype(jnp.bfloat16)
    w2T = W2.T.astype(jnp.bfloat16)
    semb = size_emb.astype(jnp.bfloat16)
    bnd = boundaries.reshape(B, 1, S)
    pos = pos_enc.reshape(C, D)

    const = lambda *_: (0, 0)
    out = pl.pallas_call(
        _fused_kernel,
        grid=(B,),
        in_specs=[
            pl.BlockSpec((1, S, D), lambda b: (b, 0, 0)),
            pl.BlockSpec((1, 1, S), lambda b: (b, 0, 0)),
            pl.BlockSpec((D, 3 * D), const),
            pl.BlockSpec((1, 3 * D), const),
            pl.BlockSpec((D, D), const),
            pl.BlockSpec((1, D), const),
            pl.BlockSpec((E, D), const),
            pl.BlockSpec((C, D), const),
            pl.BlockSpec((D, 2 * D), const),
            pl.BlockSpec((1, 2 * D), const),
            pl.BlockSpec((2 * D, D), const),
            pl.BlockSpec((1, D), const),
            pl.BlockSpec((1, D), const),
            pl.BlockSpec((1, D), const),
        ],
        out_specs=pl.BlockSpec((1, C, D), lambda b: (b, 0, 0)),
        out_shape=jax.ShapeDtypeStruct((B, C, D), jnp.float32),
    )(xb, bnd, winT, b_in.reshape(1, -1), woutT, b_out.reshape(1, -1),
      semb, pos, w1T, b1.reshape(1, -1), w2T, b2.reshape(1, -1),
      ln_g.reshape(1, -1), ln_b.reshape(1, -1))
    return out


# speedup vs baseline: 2.3023x; 1.1442x over previous
import jax
import jax.numpy as jnp
import numpy as np
from jax.experimental import pallas as pl
from jax.experimental.pallas import tpu as pltpu

B = 4
S = 512
D = 1536
H = 12
HD = D // H
C = 256
E = 1024
THRESH = 0.9
_SCALE = 1.0 / np.sqrt(HD)


def _erf(x):
    a1, a2, a3, a4, a5 = (0.254829592, -0.284496736, 1.421413741,
                          -1.453152027, 1.061405429)
    p = 0.3275911
    sgn = jnp.where(x < 0.0, -1.0, 1.0)
    ax = jnp.abs(x)
    t = 1.0 / (1.0 + p * ax)
    poly = ((((a5 * t + a4) * t + a3) * t + a2) * t + a1) * t
    y = 1.0 - poly * jnp.exp(-ax * ax)
    return sgn * y


def _gelu_exact(x):
    return 0.5 * x * (1.0 + _erf(x * np.float32(1.0 / np.sqrt(2.0))))


def _fused_kernel(x_ref, bnd_ref, winT_ref, bin_ref, woutT_ref, bout_ref,
                  semb_ref, pos_ref, w1T_ref, b1_ref, w2T_ref, b2_ref,
                  g_ref, beta_ref, out_ref):
    xb = x_ref[0]
    m = (bnd_ref[0] > THRESH).astype(jnp.float32)

    row = jax.lax.broadcasted_iota(jnp.int32, (S, S), 0)
    col = jax.lax.broadcasted_iota(jnp.int32, (S, S), 1)
    tri_low = (col <= row).astype(jnp.float32)
    cid_col = jnp.sum(tri_low * m, axis=1, keepdims=True)
    tri_up = (row <= col).astype(jnp.float32)
    cid_lane = jnp.dot(m, tri_up, preferred_element_type=jnp.float32)

    allowed = cid_col == cid_lane

    allowedf = allowed.astype(jnp.float32)

    qkv = jax.lax.dot_general(xb, winT_ref[...], (((1,), (1,)), ((), ())),
                              preferred_element_type=jnp.float32)
    qkv = qkv + bin_ref[0]
    qs = qkv[:, :D] * _SCALE

    o_parts = []
    for h in range(H):
        qh = qs[:, h * HD:(h + 1) * HD].astype(jnp.bfloat16)
        kh = qkv[:, D + h * HD:D + (h + 1) * HD].astype(jnp.bfloat16)
        vh = qkv[:, 2 * D + h * HD:2 * D + (h + 1) * HD].astype(jnp.bfloat16)
        s = jax.lax.dot_general(qh, kh, (((1,), (1,)), ((), ())),
                                preferred_element_type=jnp.float32)
        e = jnp.exp(s - jnp.max(s, axis=1, keepdims=True)) * allowedf
        a = e * (1.0 / jnp.sum(e, axis=1, keepdims=True))
        oh = jnp.dot(a.astype(jnp.bfloat16), vh,
                     preferred_element_type=jnp.float32)
        o_parts.append(oh.astype(jnp.bfloat16))
    o = jnp.concatenate(o_parts, axis=1)

    slot = 1.0 + jax.lax.broadcasted_iota(jnp.int32, (C, 1), 0).astype(jnp.float32)
    onehot = (slot == cid_lane).astype(jnp.float32)
    lens = jnp.sum(onehot, axis=1, keepdims=True)
    sums = jnp.dot(onehot.astype(jnp.bfloat16), o,
                   preferred_element_type=jnp.float32)
    means_o = sums * (1.0 / jnp.maximum(lens, 1.0))
    means = jax.lax.dot_general(means_o.astype(jnp.bfloat16), woutT_ref[...],
                                (((1,), (1,)), ((), ())),
                                preferred_element_type=jnp.float32) + bout_ref[0]

    idx = jnp.minimum(lens, float(E - 1))
    eiota = jax.lax.broadcasted_iota(jnp.int32, (1, E), 1).astype(jnp.float32)
    oneh2 = (idx == eiota).astype(jnp.bfloat16)
    sv = jnp.dot(oneh2, semb_ref[...], preferred_element_type=jnp.float32)

    r = means + sv
    r = jnp.where(lens > 0.0, r, 0.0)
    ct = r + pos_ref[...]

    h1 = jax.lax.dot_general(ct.astype(jnp.bfloat16), w1T_ref[...],
                             (((1,), (1,)), ((), ())),
                             preferred_element_type=jnp.float32) + b1_ref[0]
    h1 = _gelu_exact(h1)
    h2 = jax.lax.dot_general(h1.astype(jnp.bfloat16), w2T_ref[...],
                             (((1,), (1,)), ((), ())),
                             preferred_element_type=jnp.float32) + b2_ref[0]
    mu = jnp.mean(h2, axis=1, keepdims=True)
    var = jnp.mean((h2 - mu) * (h2 - mu), axis=1, keepdims=True)
    hn = (h2 - mu) * jax.lax.rsqrt(var + 1e-5) * g_ref[0] + beta_ref[0]
    out_ref[0] = hn


def kernel(x, boundaries, W_in, b_in, W_out, b_out, size_emb, pos_enc,
           W1, b1, W2, b2, ln_g, ln_b):
    xb = x.astype(jnp.bfloat16)
    winT = W_in.astype(jnp.bfloat16)
    woutT = W_out.astype(jnp.bfloat16)
    w1T = W1.astype(jnp.bfloat16)
    w2T = W2.astype(jnp.bfloat16)
    semb = size_emb.astype(jnp.bfloat16)
    bnd = boundaries.reshape(B, 1, S)
    pos = pos_enc.reshape(C, D)

    const = lambda *_: (0, 0)
    out = pl.pallas_call(
        _fused_kernel,
        grid=(B,),
        in_specs=[
            pl.BlockSpec((1, S, D), lambda b: (b, 0, 0)),
            pl.BlockSpec((1, 1, S), lambda b: (b, 0, 0)),
            pl.BlockSpec((3 * D, D), const),
            pl.BlockSpec((1, 3 * D), const),
            pl.BlockSpec((D, D), const),
            pl.BlockSpec((1, D), const),
            pl.BlockSpec((E, D), const),
            pl.BlockSpec((C, D), const),
            pl.BlockSpec((2 * D, D), const),
            pl.BlockSpec((1, 2 * D), const),
            pl.BlockSpec((D, 2 * D), const),
            pl.BlockSpec((1, D), const),
            pl.BlockSpec((1, D), const),
            pl.BlockSpec((1, D), const),
        ],
        out_specs=pl.BlockSpec((1, C, D), lambda b: (b, 0, 0)),
        out_shape=jax.ShapeDtypeStruct((B, C, D), jnp.float32),
    )(xb, bnd, winT, b_in.reshape(1, -1), woutT, b_out.reshape(1, -1),
      semb, pos, w1T, b1.reshape(1, -1), w2T, b2.reshape(1, -1),
      ln_g.reshape(1, -1), ln_b.reshape(1, -1))
    return out
